# baseline (speedup 1.0000x reference)
import functools

import jax
import jax.numpy as jnp
import numpy as np
from jax import lax
from jax.experimental import pallas as pl
from jax.experimental.pallas import tpu as pltpu
from jax.experimental.pallas import tpu_sc as plsc

N = 10000
E = 320000
IN_DIM = 128
HID = 32
NREL = 15
NLAYERS = 4
EPS = 1e-5

ROWW = 48
NACC = 10240
NW = 32
EP = 327680
CH = 512
KSUB = CH // 128
RPT = NACC // 16
PT0 = 18432
PT1 = 2048


def _prologue_body(d_ref, lnw_ref, lnb_ref, pos_ref, fln_ref, c_ref):
    d = d_ref[:]
    lane = lax.broadcasted_iota(jnp.int32, (N, IN_DIM), 1)
    pmask = (lane < 3).astype(jnp.float32)
    fmask = (lane >= 3).astype(jnp.float32)
    dpos = d * pmask
    mu = jnp.sum(dpos, axis=0, keepdims=True) * (1.0 / N)
    cen = (d - mu) * pmask
    mx = jnp.max(jnp.abs(cen))
    pos_s = cen * (0.999999 / mx)
    mu2 = jnp.sum(pos_s, axis=0, keepdims=True) * (1.0 / N)
    pc = (pos_s - mu2) * pmask
    c_ref[:] = lax.dot_general(pc, pc, (((0,), (0,)), ((), ())),
                               preferred_element_type=jnp.float32)
    cnt = 1.0 / (N * (IN_DIM - 3))
    fm = jnp.sum(d * fmask) * cnt
    fc = (d - fm) * fmask
    fv = jnp.sum(fc * fc) * cnt
    fln = fc * (1.0 / (jnp.sqrt(fv) + EPS)) * lnw_ref[:] + lnb_ref[:]
    pos_ref[:] = pos_s
    fln_ref[:] = fln * fmask


_prologue = pl.pallas_call(
    _prologue_body,
    out_shape=(
        jax.ShapeDtypeStruct((N, IN_DIM), jnp.float32),
        jax.ShapeDtypeStruct((N, IN_DIM), jnp.float32),
        jax.ShapeDtypeStruct((IN_DIM, IN_DIM), jnp.float32),
    ),
)


def _prep_body(x_ref, cw_ref, add_ref, cqk_ref, y1_ref, y2_ref, mx_ref):
    i = pl.program_id(0)
    x = x_ref[:]
    y1_ref[:] = lax.dot_general(x, cw_ref[:], (((1,), (0,)), ((), ())),
                                preferred_element_type=jnp.float32) + add_ref[:]
    y2 = lax.dot_general(x, cqk_ref[:], (((1,), (0,)), ((), ())),
                         preferred_element_type=jnp.float32)
    y2_ref[:] = y2
    bm = jnp.max(y2, axis=0, keepdims=True)

    @pl.when(i == 0)
    def _():
        mx_ref[:] = bm

    @pl.when(i > 0)
    def _():
        mx_ref[:] = jnp.maximum(mx_ref[:], bm)


def _make_prep(din):
    nb = 2000
    grid = (N // nb,)
    in_specs = [pl.BlockSpec((nb, din), lambda i: (i, 0))]
    in_specs += [
        pl.BlockSpec((din, NREL * ROWW), lambda i: (0, 0)),
        pl.BlockSpec((1, NREL * ROWW), lambda i: (0, 0)),
        pl.BlockSpec((din, 32), lambda i: (0, 0)),
    ]
    return pl.pallas_call(
        _prep_body,
        grid=grid,
        in_specs=in_specs,
        out_specs=(
            pl.BlockSpec((nb, NREL * ROWW), lambda i: (i, 0)),
            pl.BlockSpec((nb, 32), lambda i: (i, 0)),
            pl.BlockSpec((1, 32), lambda i: (0, 0)),
        ),
        out_shape=(
            jax.ShapeDtypeStruct((N, NREL * ROWW), jnp.float32),
            jax.ShapeDtypeStruct((N, 32), jnp.float32),
            jax.ShapeDtypeStruct((1, 32), jnp.float32),
        ),
    )


def _prep0_body(pos_ref, fln_ref, v_ref, cw_ref, add_ref, cqk_ref, y1_ref,
                y2_ref, mx_ref):
    i = pl.program_id(0)
    x = lax.dot_general(pos_ref[:], v_ref[:], (((1,), (0,)), ((), ())),
                        preferred_element_type=jnp.float32) + fln_ref[:]
    y1_ref[:] = lax.dot_general(x, cw_ref[:], (((1,), (0,)), ((), ())),
                                preferred_element_type=jnp.float32) + add_ref[:]
    y2 = lax.dot_general(x, cqk_ref[:], (((1,), (0,)), ((), ())),
                         preferred_element_type=jnp.float32)
    y2_ref[:] = y2
    bm = jnp.max(y2, axis=0, keepdims=True)

    @pl.when(i == 0)
    def _():
        mx_ref[:] = bm

    @pl.when(i > 0)
    def _():
        mx_ref[:] = jnp.maximum(mx_ref[:], bm)


def _make_prep0():
    nb = 2000
    return pl.pallas_call(
        _prep0_body,
        grid=(N // nb,),
        in_specs=[
            pl.BlockSpec((nb, IN_DIM), lambda i: (i, 0)),
            pl.BlockSpec((nb, IN_DIM), lambda i: (i, 0)),
            pl.BlockSpec((IN_DIM, IN_DIM), lambda i: (0, 0)),
            pl.BlockSpec((IN_DIM, NREL * ROWW), lambda i: (0, 0)),
            pl.BlockSpec((1, NREL * ROWW), lambda i: (0, 0)),
            pl.BlockSpec((IN_DIM, 32), lambda i: (0, 0)),
        ],
        out_specs=(
            pl.BlockSpec((nb, NREL * ROWW), lambda i: (i, 0)),
            pl.BlockSpec((nb, 32), lambda i: (i, 0)),
            pl.BlockSpec((1, 32), lambda i: (0, 0)),
        ),
        out_shape=(
            jax.ShapeDtypeStruct((N, NREL * ROWW), jnp.float32),
            jax.ShapeDtypeStruct((N, 32), jnp.float32),
            jax.ShapeDtypeStruct((1, 32), jnp.float32),
        ),
    )


def _edge_body(xw_hbm, qn_hbm, src_hbm, dst_hbm, et_hbm, cb_hbm, zv_hbm,
               outv_hbm,
               accv_sh, srcb, dstb, etb, fsrc, fdst, dsti, rows, qd,
               cbref, sem_in, sem_g, sem_s):
    cid = lax.axis_index("c")
    sid = lax.axis_index("s")

    pltpu.sync_copy(zv_hbm.at[pl.ds(sid * RPT, RPT), :],
                    accv_sh.at[pl.ds(sid * RPT, RPT), :])
    pltpu.sync_copy(cb_hbm, cbref)
    plsc.subcore_barrier()

    cbv = cbref[pl.ds(0, 16)]
    base = jnp.where(cid == 0, sid * PT0, 16 * PT0 + sid * PT1)
    nt = jnp.where(cid == 0, PT0 // (2 * CH), PT1 // (2 * CH))

    def fire_idxcopy(c, p):
        off = base + c * CH
        pltpu.async_copy(src_hbm.at[pl.ds(off, CH)], srcb.at[p], sem_in)
        pltpu.async_copy(dst_hbm.at[pl.ds(off, CH)], dstb.at[p], sem_in)
        pltpu.async_copy(et_hbm.at[pl.ds(off, CH)], etb.at[p], sem_in)

    def drain_idxcopy(p):
        for hsrc, ref in ((src_hbm, srcb), (dst_hbm, dstb), (et_hbm, etb)):
            pltpu.make_async_copy(hsrc.at[pl.ds(0, CH)], ref.at[p],
                                  sem_in).wait()

    def idx_gather(p):
        @pl.loop(0, CH // 16)
        def _(i):
            s = srcb[p, pl.ds(i * 16, 16)]
            e = etb[p, pl.ds(i * 16, 16)]
            d = dstb[p, pl.ds(i * 16, 16)]
            fsrc[p, pl.ds(i * 16, 16)] = s * NREL + e
            fdst[p, pl.ds(i * 16, 16)] = d * NREL + e

    def idx_scatter(p):
        @pl.loop(0, CH // 16)
        def _(i):
            d = dstb[p, pl.ds(i * 16, 16)]
            r, col = i // 8, (i % 8) * 16
            dsti[p, r, pl.ds(col, 16)] = d

    def fire_gathers(p):
        pltpu.async_copy(xw_hbm.at[fsrc.at[p]], rows.at[p], sem_g)
        pltpu.async_copy(qn_hbm.at[fdst.at[p]], qd.at[p], sem_g)

    def drain_gathers(p):
        pltpu.make_async_copy(xw_hbm.at[pl.ds(0, CH)], rows.at[p],
                              sem_g).wait()
        pltpu.make_async_copy(qn_hbm.at[pl.ds(0, CH)], qd.at[p],
                              sem_g).wait()

    def fire_scatters(p):
        for k in range(KSUB):
            pltpu.async_copy(rows.at[p, pl.ds(k * 128, 128), :],
                             accv_sh.at[dsti.at[p, k]], sem_s, add=True)

    def drain_scatters(p):
        for k in range(KSUB):
            pltpu.make_async_copy(rows.at[p, pl.ds(k * 128, 128), :],
                                  accv_sh.at[pl.ds(0, 128), :], sem_s).wait()

    def w_scale(p):
        iota16 = lax.broadcasted_iota(jnp.int32, (16,), 0)
        col32 = jnp.full((16,), 32, jnp.int32)

        @pl.loop(0, CH // 16)
        def _(i):
            eids = i * 16 + iota16
            ksv = plsc.load_gather(rows.at[p], [eids, jnp.full((16,), 33,
                                                               jnp.int32)])
            a = qd[p, pl.ds(i * 16, 16)] + ksv
            a = jnp.where(a >= 0.0, a, 0.2 * a) - cbv
            wv = jnp.exp(a)
            plsc.store_scatter(rows.at[p], [eids, col32], wv)
            for j in range(16):
                wj = wv[j]
                r = i * 16 + j
                rows[p, r, pl.ds(0, 16)] = rows[p, r, pl.ds(0, 16)] * wj
                rows[p, r, pl.ds(16, 16)] = rows[p, r, pl.ds(16, 16)] * wj

    fire_idxcopy(0, 0)
    drain_idxcopy(0)
    idx_gather(0)
    idx_scatter(0)
    fire_gathers(0)
    fire_idxcopy(1, 1)

    @pl.loop(0, nt)
    def _pair(t):
        a = 2 * t

        @pl.when(t > 0)
        def _():
            drain_scatters(1)

        drain_idxcopy(1)
        idx_gather(1)
        idx_scatter(1)

        @pl.when(t < nt - 1)
        def _():
            fire_idxcopy(a + 2, 0)

        drain_gathers(0)
        fire_gathers(1)
        w_scale(0)
        fire_scatters(0)

        @pl.when(t < nt - 1)
        def _():
            drain_idxcopy(0)
            idx_gather(0)

        drain_gathers(1)
        drain_scatters(0)

        @pl.when(t < nt - 1)
        def _():
            idx_scatter(0)
            fire_gathers(0)
            fire_idxcopy(a + 3, 1)

        w_scale(1)
        fire_scatters(1)

    drain_scatters(1)
    plsc.subcore_barrier()
    pltpu.sync_copy(accv_sh.at[pl.ds(sid * RPT, RPT), :],
                    outv_hbm.at[cid, pl.ds(sid * RPT, RPT), :])


_EDGE = None


def _get_edge_kernel():
    global _EDGE
    if _EDGE is None:
        _EDGE = functools.partial(
            pl.kernel,
            out_type=jax.ShapeDtypeStruct((2, NACC, ROWW), jnp.float32),
            mesh=plsc.VectorSubcoreMesh(core_axis_name="c",
                                        subcore_axis_name="s",
                                        num_cores=2, num_subcores=16),
            compiler_params=pltpu.CompilerParams(use_tc_tiling_on_sc=False,
                                                 needs_layout_passes=False),
            scratch_types=[
                pltpu.VMEM_SHARED((NACC, ROWW), jnp.float32),
                pltpu.VMEM((2, CH), jnp.int32),
                pltpu.VMEM((2, CH), jnp.int32),
                pltpu.VMEM((2, CH), jnp.int32),
                pltpu.VMEM((2, CH), jnp.int32),
                pltpu.VMEM((2, CH), jnp.int32),
                pltpu.VMEM((2, KSUB, 128), jnp.int32),
                pltpu.VMEM((2, CH, ROWW), jnp.float32),
                pltpu.VMEM((2, CH), jnp.float32),
                pltpu.VMEM((16,), jnp.float32),
                pltpu.SemaphoreType.DMA,
                pltpu.SemaphoreType.DMA,
                pltpu.SemaphoreType.DMA,
            ],
        )(_edge_body)
    return _EDGE


def _finish_body(av_ref, b_ref, prev_ref, bnw_ref, bnb_ref, x_ref):
    s = av_ref[0] + av_ref[1]
    lane = lax.broadcasted_iota(jnp.int32, (N, ROWW), 1)
    m32 = (lane < HID).astype(jnp.float32)
    denom = jnp.sum(jnp.where(lane == HID, s, 0.0), axis=1, keepdims=True)
    h = (s * (1.0 / (denom + 1e-16)) + b_ref[:] + prev_ref[:]) * m32
    cnt = 1.0 / (N * HID)
    m = jnp.sum(h) * cnt
    xc = (h - m) * m32
    v = jnp.sum(xc * xc) * cnt
    xn = xc * (1.0 / (jnp.sqrt(v) + EPS)) * bnw_ref[:] + bnb_ref[:]
    x_ref[:] = xn * (1.0 / (1.0 + jnp.exp(-xn)))


_finish = pl.pallas_call(
    _finish_body,
    out_shape=jax.ShapeDtypeStruct((N, ROWW), jnp.float32),
)


def _epilogue_body(x_ref, w_ref, b_ref, o_ref):
    x3 = jnp.sum(x_ref[:], axis=0, keepdims=True) * (1.0 / N)
    y = lax.dot_general(x3, w_ref[:], (((1,), (0,)), ((), ())),
                        preferred_element_type=jnp.float32) + b_ref[:]
    o_ref[:] = y * (1.0 / (1.0 + jnp.exp(-y)))


_epilogue = pl.pallas_call(
    _epilogue_body,
    out_shape=jax.ShapeDtypeStruct((1, 256), jnp.float32),
)


def _pack_weights(params):
    packed = {}
    for i in range(NLAYERS):
        W = params['conv%d_w' % i]
        D = W.shape[1]
        Dp = D if i == 0 else ROWW
        qv = jnp.einsum('rdo,o->dr', W, params['conv%d_q' % i][:, 0])
        kv = jnp.einsum('rdo,o->dr', W, params['conv%d_k' % i][:, 0])
        Wt = jnp.transpose(W, (1, 0, 2))
        Wt = jnp.pad(Wt, ((0, Dp - D), (0, 0), (0, ROWW - HID)))
        Wt = Wt.at[:D, :, 33].set(kv)
        packed['cw%d' % i] = Wt.reshape(Dp, NREL * ROWW)
        add = np.zeros((1, NREL * ROWW), np.float32)
        add[0, HID::ROWW] = 1.0
        packed['add'] = jnp.asarray(add)
        cqk = jnp.zeros((Dp, 32), jnp.float32)
        cqk = cqk.at[:D, :NREL].set(qv).at[:D, 16:16 + NREL].set(kv)
        packed['cqk%d' % i] = cqk
        packed['b%d' % i] = jnp.pad(params['conv%d_b' % i],
                                    (0, ROWW - HID)).reshape(1, ROWW)
        packed['bnw%d' % i] = jnp.pad(params['bn%d_w' % i],
                                      (0, ROWW - HID)).reshape(1, ROWW)
        packed['bnb%d' % i] = jnp.pad(params['bn%d_b' % i],
                                      (0, ROWW - HID)).reshape(1, ROWW)
    packed['lnw'] = jnp.pad(params['ln1_w'], (3, 0)).reshape(1, IN_DIM)
    packed['lnb'] = jnp.pad(params['ln1_b'], (3, 0)).reshape(1, IN_DIM)
    packed['linw'] = jnp.pad(params['lin1_w'].T, ((0, ROWW - HID), (0, 0)))
    packed['linb'] = params['lin1_b'].reshape(1, 256)
    return packed


_PREP = None


def _get_preps():
    global _PREP
    if _PREP is None:
        _PREP = (_make_prep0(), _make_prep(ROWW))
    return _PREP


def kernel(data_batch_1, data_batch_2, edge_index_1, edge_index_2,
           edge_type_1, edge_type_2, params):
    del data_batch_2, edge_index_2, edge_type_2
    pk = _pack_weights(params)
    prep0, prep = _get_preps()

    pos_s, fln, C = _prologue(data_batch_1, pk['lnw'], pk['lnb'])
    _, v = jnp.linalg.eigh(C[:3, :3])
    vpad = jnp.zeros((IN_DIM, IN_DIM), jnp.float32).at[:3, :3].set(v)

    src = jnp.pad(edge_index_1[0], (0, EP - E))
    dst = jnp.pad(edge_index_1[1], (0, EP - E), constant_values=N)
    et = jnp.pad(edge_type_1, (0, EP - E))
    zv = jnp.zeros((NACC, ROWW), jnp.float32)

    x = None
    for i in range(NLAYERS):
        if i == 0:
            y1, y2, mx = prep0(pos_s, fln, vpad, pk['cw0'], pk['add'],
                               pk['cqk0'])
        else:
            y1, y2, mx = prep(x, pk['cw%d' % i], pk['add'], pk['cqk%d' % i])
        xw = y1.reshape(N * NREL, ROWW)
        qn = jnp.concatenate([y2[:, :NREL].reshape(N * NREL),
                              jnp.full((16,), -1e30, jnp.float32)])
        z = jnp.max(mx[0, :NREL]) + jnp.max(mx[0, 16:16 + NREL])
        cb = jnp.where(z >= 0.0, z, 0.2 * z)
        cb16 = jnp.broadcast_to(cb.reshape(1), (16,))
        outv = _get_edge_kernel()(xw, qn, src, dst, et, cb16, zv)
        prev = x[:, :] if x is not None else jnp.zeros((N, ROWW), jnp.float32)
        x = _finish(outv[:, :N, :], pk['b%d' % i], prev, pk['bnw%d' % i],
                    pk['bnb%d' % i])

    out = _epilogue(x, pk['linw'], pk['linb'])
    return out, out


# speedup vs baseline: 1.0008x; 1.0008x over previous
import functools

import jax
import jax.numpy as jnp
import numpy as np
from jax import lax
from jax.experimental import pallas as pl
from jax.experimental.pallas import tpu as pltpu
from jax.experimental.pallas import tpu_sc as plsc

N = 10000
E = 320000
IN_DIM = 128
HID = 32
NREL = 15
NLAYERS = 4
EPS = 1e-5

ROWW = 48
NACC = 10240
NW = 32
EP = 327680
CH = 512
KSUB = CH // 128
RPT = NACC // 16
PT0 = 18432
PT1 = 2048


def _prologue_body(d_ref, lnw_ref, lnb_ref, pos_ref, fln_ref, c_ref):
    d = d_ref[:]
    lane = lax.broadcasted_iota(jnp.int32, (N, IN_DIM), 1)
    pmask = (lane < 3).astype(jnp.float32)
    fmask = (lane >= 3).astype(jnp.float32)
    dpos = d * pmask
    mu = jnp.sum(dpos, axis=0, keepdims=True) * (1.0 / N)
    cen = (d - mu) * pmask
    mx = jnp.max(jnp.abs(cen))
    pos_s = cen * (0.999999 / mx)
    mu2 = jnp.sum(pos_s, axis=0, keepdims=True) * (1.0 / N)
    pc = (pos_s - mu2) * pmask
    c_ref[:] = lax.dot_general(pc, pc, (((0,), (0,)), ((), ())),
                               preferred_element_type=jnp.float32)
    cnt = 1.0 / (N * (IN_DIM - 3))
    fm = jnp.sum(d * fmask) * cnt
    fc = (d - fm) * fmask
    fv = jnp.sum(fc * fc) * cnt
    fln = fc * (1.0 / (jnp.sqrt(fv) + EPS)) * lnw_ref[:] + lnb_ref[:]
    pos_ref[:] = pos_s
    fln_ref[:] = fln * fmask


_prologue = pl.pallas_call(
    _prologue_body,
    out_shape=(
        jax.ShapeDtypeStruct((N, IN_DIM), jnp.float32),
        jax.ShapeDtypeStruct((N, IN_DIM), jnp.float32),
        jax.ShapeDtypeStruct((IN_DIM, IN_DIM), jnp.float32),
    ),
)


def _prep_body(x_ref, cw_ref, add_ref, cqk_ref, y1_ref, y2_ref, mx_ref):
    i = pl.program_id(0)
    x = x_ref[:]
    y1_ref[:] = lax.dot_general(x, cw_ref[:], (((1,), (0,)), ((), ())),
                                preferred_element_type=jnp.float32) + add_ref[:]
    y2 = lax.dot_general(x, cqk_ref[:], (((1,), (0,)), ((), ())),
                         preferred_element_type=jnp.float32)
    y2_ref[:] = y2
    bm = jnp.max(y2, axis=0, keepdims=True)

    @pl.when(i == 0)
    def _():
        mx_ref[:] = bm

    @pl.when(i > 0)
    def _():
        mx_ref[:] = jnp.maximum(mx_ref[:], bm)


def _make_prep(din):
    nb = 2000
    grid = (N // nb,)
    in_specs = [pl.BlockSpec((nb, din), lambda i: (i, 0))]
    in_specs += [
        pl.BlockSpec((din, NREL * ROWW), lambda i: (0, 0)),
        pl.BlockSpec((1, NREL * ROWW), lambda i: (0, 0)),
        pl.BlockSpec((din, 32), lambda i: (0, 0)),
    ]
    return pl.pallas_call(
        _prep_body,
        grid=grid,
        in_specs=in_specs,
        out_specs=(
            pl.BlockSpec((nb, NREL * ROWW), lambda i: (i, 0)),
            pl.BlockSpec((nb, 32), lambda i: (i, 0)),
            pl.BlockSpec((1, 32), lambda i: (0, 0)),
        ),
        out_shape=(
            jax.ShapeDtypeStruct((N, NREL * ROWW), jnp.float32),
            jax.ShapeDtypeStruct((N, 32), jnp.float32),
            jax.ShapeDtypeStruct((1, 32), jnp.float32),
        ),
    )


def _prep0_body(pos_ref, fln_ref, v_ref, cw_ref, add_ref, cqk_ref, y1_ref,
                y2_ref, mx_ref):
    i = pl.program_id(0)
    x = lax.dot_general(pos_ref[:], v_ref[:], (((1,), (0,)), ((), ())),
                        preferred_element_type=jnp.float32) + fln_ref[:]
    y1_ref[:] = lax.dot_general(x, cw_ref[:], (((1,), (0,)), ((), ())),
                                preferred_element_type=jnp.float32) + add_ref[:]
    y2 = lax.dot_general(x, cqk_ref[:], (((1,), (0,)), ((), ())),
                         preferred_element_type=jnp.float32)
    y2_ref[:] = y2
    bm = jnp.max(y2, axis=0, keepdims=True)

    @pl.when(i == 0)
    def _():
        mx_ref[:] = bm

    @pl.when(i > 0)
    def _():
        mx_ref[:] = jnp.maximum(mx_ref[:], bm)


def _make_prep0():
    nb = 2000
    return pl.pallas_call(
        _prep0_body,
        grid=(N // nb,),
        in_specs=[
            pl.BlockSpec((nb, IN_DIM), lambda i: (i, 0)),
            pl.BlockSpec((nb, IN_DIM), lambda i: (i, 0)),
            pl.BlockSpec((IN_DIM, IN_DIM), lambda i: (0, 0)),
            pl.BlockSpec((IN_DIM, NREL * ROWW), lambda i: (0, 0)),
            pl.BlockSpec((1, NREL * ROWW), lambda i: (0, 0)),
            pl.BlockSpec((IN_DIM, 32), lambda i: (0, 0)),
        ],
        out_specs=(
            pl.BlockSpec((nb, NREL * ROWW), lambda i: (i, 0)),
            pl.BlockSpec((nb, 32), lambda i: (i, 0)),
            pl.BlockSpec((1, 32), lambda i: (0, 0)),
        ),
        out_shape=(
            jax.ShapeDtypeStruct((N, NREL * ROWW), jnp.float32),
            jax.ShapeDtypeStruct((N, 32), jnp.float32),
            jax.ShapeDtypeStruct((1, 32), jnp.float32),
        ),
    )


def _edge_body(xw_hbm, qn_hbm, src_hbm, dst_hbm, et_hbm, cb_hbm, zv_hbm,
               outv_hbm,
               accv_sh, srcb, dstb, etb, fsrc, fdst, dsti, rows, qd,
               cbref, sem_in, sem_g, sem_s):
    cid = lax.axis_index("c")
    sid = lax.axis_index("s")

    pltpu.sync_copy(zv_hbm.at[pl.ds(sid * RPT, RPT), :],
                    accv_sh.at[pl.ds(sid * RPT, RPT), :])
    pltpu.sync_copy(cb_hbm, cbref)
    plsc.subcore_barrier()

    cbv = cbref[pl.ds(0, 16)]
    base = jnp.where(cid == 0, sid * PT0, 16 * PT0 + sid * PT1)
    nt = jnp.where(cid == 0, PT0 // (2 * CH), PT1 // (2 * CH))

    def fire_idxcopy(c, p):
        off = base + c * CH
        pltpu.async_copy(src_hbm.at[pl.ds(off, CH)], srcb.at[p], sem_in)
        pltpu.async_copy(dst_hbm.at[pl.ds(off, CH)], dstb.at[p], sem_in)
        pltpu.async_copy(et_hbm.at[pl.ds(off, CH)], etb.at[p], sem_in)

    def drain_idxcopy(p):
        for hsrc, ref in ((src_hbm, srcb), (dst_hbm, dstb), (et_hbm, etb)):
            pltpu.make_async_copy(hsrc.at[pl.ds(0, CH)], ref.at[p],
                                  sem_in).wait()

    def idx_gather(p):
        @plsc.parallel_loop(0, CH // 16, unroll=4)
        def _(i):
            s = srcb[p, pl.ds(i * 16, 16)]
            e = etb[p, pl.ds(i * 16, 16)]
            d = dstb[p, pl.ds(i * 16, 16)]
            fsrc[p, pl.ds(i * 16, 16)] = s * NREL + e
            fdst[p, pl.ds(i * 16, 16)] = d * NREL + e

    def idx_scatter(p):
        @plsc.parallel_loop(0, CH // 16, unroll=4)
        def _(i):
            d = dstb[p, pl.ds(i * 16, 16)]
            r, col = i // 8, (i % 8) * 16
            dsti[p, r, pl.ds(col, 16)] = d

    def fire_gathers(p):
        pltpu.async_copy(xw_hbm.at[fsrc.at[p]], rows.at[p], sem_g)
        pltpu.async_copy(qn_hbm.at[fdst.at[p]], qd.at[p], sem_g)

    def drain_gathers(p):
        pltpu.make_async_copy(xw_hbm.at[pl.ds(0, CH)], rows.at[p],
                              sem_g).wait()
        pltpu.make_async_copy(qn_hbm.at[pl.ds(0, CH)], qd.at[p],
                              sem_g).wait()

    def fire_scatters(p):
        for k in range(KSUB):
            pltpu.async_copy(rows.at[p, pl.ds(k * 128, 128), :],
                             accv_sh.at[dsti.at[p, k]], sem_s, add=True)

    def drain_scatters(p):
        for k in range(KSUB):
            pltpu.make_async_copy(rows.at[p, pl.ds(k * 128, 128), :],
                                  accv_sh.at[pl.ds(0, 128), :], sem_s).wait()

    def w_scale(p):
        iota16 = lax.broadcasted_iota(jnp.int32, (16,), 0)
        col32 = jnp.full((16,), 32, jnp.int32)

        @plsc.parallel_loop(0, CH // 16, unroll=2)
        def _(i):
            eids = i * 16 + iota16
            ksv = plsc.load_gather(rows.at[p], [eids, jnp.full((16,), 33,
                                                               jnp.int32)])
            a = qd[p, pl.ds(i * 16, 16)] + ksv
            a = jnp.where(a >= 0.0, a, 0.2 * a) - cbv
            wv = jnp.exp(a)
            plsc.store_scatter(rows.at[p], [eids, col32], wv)
            for j in range(16):
                wj = wv[j]
                r = i * 16 + j
                rows[p, r, pl.ds(0, 16)] = rows[p, r, pl.ds(0, 16)] * wj
                rows[p, r, pl.ds(16, 16)] = rows[p, r, pl.ds(16, 16)] * wj

    fire_idxcopy(0, 0)
    drain_idxcopy(0)
    idx_gather(0)
    idx_scatter(0)
    fire_gathers(0)
    fire_idxcopy(1, 1)

    @pl.loop(0, nt)
    def _pair(t):
        a = 2 * t

        @pl.when(t > 0)
        def _():
            drain_scatters(1)

        drain_idxcopy(1)
        idx_gather(1)
        idx_scatter(1)

        @pl.when(t < nt - 1)
        def _():
            fire_idxcopy(a + 2, 0)

        drain_gathers(0)
        fire_gathers(1)
        w_scale(0)
        fire_scatters(0)

        @pl.when(t < nt - 1)
        def _():
            drain_idxcopy(0)
            idx_gather(0)

        drain_gathers(1)
        drain_scatters(0)

        @pl.when(t < nt - 1)
        def _():
            idx_scatter(0)
            fire_gathers(0)
            fire_idxcopy(a + 3, 1)

        w_scale(1)
        fire_scatters(1)

    drain_scatters(1)
    plsc.subcore_barrier()
    pltpu.sync_copy(accv_sh.at[pl.ds(sid * RPT, RPT), :],
                    outv_hbm.at[cid, pl.ds(sid * RPT, RPT), :])


_EDGE = None


def _get_edge_kernel():
    global _EDGE
    if _EDGE is None:
        _EDGE = functools.partial(
            pl.kernel,
            out_type=jax.ShapeDtypeStruct((2, NACC, ROWW), jnp.float32),
            mesh=plsc.VectorSubcoreMesh(core_axis_name="c",
                                        subcore_axis_name="s",
                                        num_cores=2, num_subcores=16),
            compiler_params=pltpu.CompilerParams(use_tc_tiling_on_sc=False,
                                                 needs_layout_passes=False),
            scratch_types=[
                pltpu.VMEM_SHARED((NACC, ROWW), jnp.float32),
                pltpu.VMEM((2, CH), jnp.int32),
                pltpu.VMEM((2, CH), jnp.int32),
                pltpu.VMEM((2, CH), jnp.int32),
                pltpu.VMEM((2, CH), jnp.int32),
                pltpu.VMEM((2, CH), jnp.int32),
                pltpu.VMEM((2, KSUB, 128), jnp.int32),
                pltpu.VMEM((2, CH, ROWW), jnp.float32),
                pltpu.VMEM((2, CH), jnp.float32),
                pltpu.VMEM((16,), jnp.float32),
                pltpu.SemaphoreType.DMA,
                pltpu.SemaphoreType.DMA,
                pltpu.SemaphoreType.DMA,
            ],
        )(_edge_body)
    return _EDGE


def _finish_body(av_ref, b_ref, prev_ref, bnw_ref, bnb_ref, x_ref):
    s = av_ref[0] + av_ref[1]
    lane = lax.broadcasted_iota(jnp.int32, (N, ROWW), 1)
    m32 = (lane < HID).astype(jnp.float32)
    denom = jnp.sum(jnp.where(lane == HID, s, 0.0), axis=1, keepdims=True)
    h = (s * (1.0 / (denom + 1e-16)) + b_ref[:] + prev_ref[:]) * m32
    cnt = 1.0 / (N * HID)
    m = jnp.sum(h) * cnt
    xc = (h - m) * m32
    v = jnp.sum(xc * xc) * cnt
    xn = xc * (1.0 / (jnp.sqrt(v) + EPS)) * bnw_ref[:] + bnb_ref[:]
    x_ref[:] = xn * (1.0 / (1.0 + jnp.exp(-xn)))


_finish = pl.pallas_call(
    _finish_body,
    out_shape=jax.ShapeDtypeStruct((N, ROWW), jnp.float32),
)


def _epilogue_body(x_ref, w_ref, b_ref, o_ref):
    x3 = jnp.sum(x_ref[:], axis=0, keepdims=True) * (1.0 / N)
    y = lax.dot_general(x3, w_ref[:], (((1,), (0,)), ((), ())),
                        preferred_element_type=jnp.float32) + b_ref[:]
    o_ref[:] = y * (1.0 / (1.0 + jnp.exp(-y)))


_epilogue = pl.pallas_call(
    _epilogue_body,
    out_shape=jax.ShapeDtypeStruct((1, 256), jnp.float32),
)


def _pack_weights(params):
    packed = {}
    for i in range(NLAYERS):
        W = params['conv%d_w' % i]
        D = W.shape[1]
        Dp = D if i == 0 else ROWW
        qv = jnp.einsum('rdo,o->dr', W, params['conv%d_q' % i][:, 0])
        kv = jnp.einsum('rdo,o->dr', W, params['conv%d_k' % i][:, 0])
        Wt = jnp.transpose(W, (1, 0, 2))
        Wt = jnp.pad(Wt, ((0, Dp - D), (0, 0), (0, ROWW - HID)))
        Wt = Wt.at[:D, :, 33].set(kv)
        packed['cw%d' % i] = Wt.reshape(Dp, NREL * ROWW)
        add = np.zeros((1, NREL * ROWW), np.float32)
        add[0, HID::ROWW] = 1.0
        packed['add'] = jnp.asarray(add)
        cqk = jnp.zeros((Dp, 32), jnp.float32)
        cqk = cqk.at[:D, :NREL].set(qv).at[:D, 16:16 + NREL].set(kv)
        packed['cqk%d' % i] = cqk
        packed['b%d' % i] = jnp.pad(params['conv%d_b' % i],
                                    (0, ROWW - HID)).reshape(1, ROWW)
        packed['bnw%d' % i] = jnp.pad(params['bn%d_w' % i],
                                      (0, ROWW - HID)).reshape(1, ROWW)
        packed['bnb%d' % i] = jnp.pad(params['bn%d_b' % i],
                                      (0, ROWW - HID)).reshape(1, ROWW)
    packed['lnw'] = jnp.pad(params['ln1_w'], (3, 0)).reshape(1, IN_DIM)
    packed['lnb'] = jnp.pad(params['ln1_b'], (3, 0)).reshape(1, IN_DIM)
    packed['linw'] = jnp.pad(params['lin1_w'].T, ((0, ROWW - HID), (0, 0)))
    packed['linb'] = params['lin1_b'].reshape(1, 256)
    return packed


_PREP = None


def _get_preps():
    global _PREP
    if _PREP is None:
        _PREP = (_make_prep0(), _make_prep(ROWW))
    return _PREP


def kernel(data_batch_1, data_batch_2, edge_index_1, edge_index_2,
           edge_type_1, edge_type_2, params):
    del data_batch_2, edge_index_2, edge_type_2
    pk = _pack_weights(params)
    prep0, prep = _get_preps()

    pos_s, fln, C = _prologue(data_batch_1, pk['lnw'], pk['lnb'])
    _, v = jnp.linalg.eigh(C[:3, :3])
    vpad = jnp.zeros((IN_DIM, IN_DIM), jnp.float32).at[:3, :3].set(v)

    src = jnp.pad(edge_index_1[0], (0, EP - E))
    dst = jnp.pad(edge_index_1[1], (0, EP - E), constant_values=N)
    et = jnp.pad(edge_type_1, (0, EP - E))
    zv = jnp.zeros((NACC, ROWW), jnp.float32)

    x = None
    for i in range(NLAYERS):
        if i == 0:
            y1, y2, mx = prep0(pos_s, fln, vpad, pk['cw0'], pk['add'],
                               pk['cqk0'])
        else:
            y1, y2, mx = prep(x, pk['cw%d' % i], pk['add'], pk['cqk%d' % i])
        xw = y1.reshape(N * NREL, ROWW)
        qn = jnp.concatenate([y2[:, :NREL].reshape(N * NREL),
                              jnp.full((16,), -1e30, jnp.float32)])
        z = jnp.max(mx[0, :NREL]) + jnp.max(mx[0, 16:16 + NREL])
        cb = jnp.where(z >= 0.0, z, 0.2 * z)
        cb16 = jnp.broadcast_to(cb.reshape(1), (16,))
        outv = _get_edge_kernel()(xw, qn, src, dst, et, cb16, zv)
        prev = x[:, :] if x is not None else jnp.zeros((N, ROWW), jnp.float32)
        x = _finish(outv[:, :N, :], pk['b%d' % i], prev, pk['bnw%d' % i],
                    pk['bnb%d' % i])

    out = _epilogue(x, pk['linw'], pk['linb'])
    return out, out


# speedup vs baseline: 1.1180x; 1.1171x over previous
import functools

import jax
import jax.numpy as jnp
import numpy as np
from jax import lax
from jax.experimental import pallas as pl
from jax.experimental.pallas import tpu as pltpu
from jax.experimental.pallas import tpu_sc as plsc

N = 10000
E = 320000
IN_DIM = 128
HID = 32
NREL = 15
NLAYERS = 4
EPS = 1e-5

NACC = 10240
EP = 327680
CH = 512
KSUB = CH // 128
RPT = NACC // 16
PT0 = 18432
PT1 = 2048
WB = NREL * HID


def _prologue_body(d_ref, lnw_ref, lnb_ref, pos_ref, fln_ref, c_ref):
    d = d_ref[:]
    lane = lax.broadcasted_iota(jnp.int32, (N, IN_DIM), 1)
    pmask = (lane < 3).astype(jnp.float32)
    fmask = (lane >= 3).astype(jnp.float32)
    dpos = d * pmask
    mu = jnp.sum(dpos, axis=0, keepdims=True) * (1.0 / N)
    cen = (d - mu) * pmask
    mx = jnp.max(jnp.abs(cen))
    pos_s = cen * (0.999999 / mx)
    mu2 = jnp.sum(pos_s, axis=0, keepdims=True) * (1.0 / N)
    pc = (pos_s - mu2) * pmask
    c_ref[:] = lax.dot_general(pc, pc, (((0,), (0,)), ((), ())),
                               preferred_element_type=jnp.float32)
    cnt = 1.0 / (N * (IN_DIM - 3))
    fm = jnp.sum(d * fmask) * cnt
    fc = (d - fm) * fmask
    fv = jnp.sum(fc * fc) * cnt
    fln = fc * (1.0 / (jnp.sqrt(fv) + EPS)) * lnw_ref[:] + lnb_ref[:]
    pos_ref[:] = pos_s
    fln_ref[:] = fln * fmask


_prologue = pl.pallas_call(
    _prologue_body,
    out_shape=(
        jax.ShapeDtypeStruct((N, IN_DIM), jnp.float32),
        jax.ShapeDtypeStruct((N, IN_DIM), jnp.float32),
        jax.ShapeDtypeStruct((IN_DIM, IN_DIM), jnp.float32),
    ),
)


def _prep_body(x_ref, cw_ref, cqk_ref, y1_ref, y2_ref, mx_ref):
    i = pl.program_id(0)
    x = x_ref[:]
    y1_ref[:] = lax.dot_general(x, cw_ref[:], (((1,), (0,)), ((), ())),
                                preferred_element_type=jnp.float32)
    y2 = lax.dot_general(x, cqk_ref[:], (((1,), (0,)), ((), ())),
                         preferred_element_type=jnp.float32)
    y2_ref[:] = y2
    bm = jnp.max(y2, axis=0, keepdims=True)

    @pl.when(i == 0)
    def _():
        mx_ref[:] = bm

    @pl.when(i > 0)
    def _():
        mx_ref[:] = jnp.maximum(mx_ref[:], bm)


def _make_prep(din):
    nb = 2000
    return pl.pallas_call(
        _prep_body,
        grid=(N // nb,),
        in_specs=[
            pl.BlockSpec((nb, din), lambda i: (i, 0)),
            pl.BlockSpec((din, WB), lambda i: (0, 0)),
            pl.BlockSpec((din, 32), lambda i: (0, 0)),
        ],
        out_specs=(
            pl.BlockSpec((nb, WB), lambda i: (i, 0)),
            pl.BlockSpec((nb, 32), lambda i: (i, 0)),
            pl.BlockSpec((1, 32), lambda i: (0, 0)),
        ),
        out_shape=(
            jax.ShapeDtypeStruct((N, WB), jnp.float32),
            jax.ShapeDtypeStruct((N, 32), jnp.float32),
            jax.ShapeDtypeStruct((1, 32), jnp.float32),
        ),
    )


def _prep0_body(pos_ref, fln_ref, v_ref, cw_ref, cqk_ref, y1_ref, y2_ref,
                mx_ref):
    i = pl.program_id(0)
    x = lax.dot_general(pos_ref[:], v_ref[:], (((1,), (0,)), ((), ())),
                        preferred_element_type=jnp.float32) + fln_ref[:]
    y1_ref[:] = lax.dot_general(x, cw_ref[:], (((1,), (0,)), ((), ())),
                                preferred_element_type=jnp.float32)
    y2 = lax.dot_general(x, cqk_ref[:], (((1,), (0,)), ((), ())),
                         preferred_element_type=jnp.float32)
    y2_ref[:] = y2
    bm = jnp.max(y2, axis=0, keepdims=True)

    @pl.when(i == 0)
    def _():
        mx_ref[:] = bm

    @pl.when(i > 0)
    def _():
        mx_ref[:] = jnp.maximum(mx_ref[:], bm)


def _make_prep0():
    nb = 2000
    return pl.pallas_call(
        _prep0_body,
        grid=(N // nb,),
        in_specs=[
            pl.BlockSpec((nb, IN_DIM), lambda i: (i, 0)),
            pl.BlockSpec((nb, IN_DIM), lambda i: (i, 0)),
            pl.BlockSpec((IN_DIM, IN_DIM), lambda i: (0, 0)),
            pl.BlockSpec((IN_DIM, WB), lambda i: (0, 0)),
            pl.BlockSpec((IN_DIM, 32), lambda i: (0, 0)),
        ],
        out_specs=(
            pl.BlockSpec((nb, WB), lambda i: (i, 0)),
            pl.BlockSpec((nb, 32), lambda i: (i, 0)),
            pl.BlockSpec((1, 32), lambda i: (0, 0)),
        ),
        out_shape=(
            jax.ShapeDtypeStruct((N, WB), jnp.float32),
            jax.ShapeDtypeStruct((N, 32), jnp.float32),
            jax.ShapeDtypeStruct((1, 32), jnp.float32),
        ),
    )


def _edge_body(xw_hbm, qn_hbm, kn_hbm, src_hbm, dst_hbm, et_hbm, cb_hbm,
               zv_hbm, zs_hbm,
               outv_hbm, outs_hbm,
               accv_sh, accs_sh, srcb, dstb, etb, fsrc, fdst, dsti, rows,
               qd, ks, wb, cbref, sem_in, sem_g, sem_s):
    cid = lax.axis_index("c")
    sid = lax.axis_index("s")

    pltpu.sync_copy(zv_hbm.at[pl.ds(sid * RPT, RPT), :],
                    accv_sh.at[pl.ds(sid * RPT, RPT), :])
    pltpu.sync_copy(zs_hbm.at[pl.ds(sid * RPT, RPT)],
                    accs_sh.at[pl.ds(sid * RPT, RPT)])
    pltpu.sync_copy(cb_hbm, cbref)
    plsc.subcore_barrier()

    cbv = cbref[pl.ds(0, 16)]
    base = jnp.where(cid == 0, sid * PT0, 16 * PT0 + sid * PT1)
    nt = jnp.where(cid == 0, PT0 // (2 * CH), PT1 // (2 * CH))

    def fire_idxcopy(c, p):
        off = base + c * CH
        pltpu.async_copy(src_hbm.at[pl.ds(off, CH)], srcb.at[p], sem_in)
        pltpu.async_copy(dst_hbm.at[pl.ds(off, CH)], dstb.at[p], sem_in)
        pltpu.async_copy(et_hbm.at[pl.ds(off, CH)], etb.at[p], sem_in)

    def drain_idxcopy(p):
        for hsrc, ref in ((src_hbm, srcb), (dst_hbm, dstb), (et_hbm, etb)):
            pltpu.make_async_copy(hsrc.at[pl.ds(0, CH)], ref.at[p],
                                  sem_in).wait()

    def idx_gather(p):
        @plsc.parallel_loop(0, CH // 16, unroll=4)
        def _(i):
            s = srcb[p, pl.ds(i * 16, 16)]
            e = etb[p, pl.ds(i * 16, 16)]
            d = dstb[p, pl.ds(i * 16, 16)]
            fsrc[p, pl.ds(i * 16, 16)] = s * NREL + e
            fdst[p, pl.ds(i * 16, 16)] = d * NREL + e

    def idx_scatter(p):
        @plsc.parallel_loop(0, CH // 16, unroll=4)
        def _(i):
            d = dstb[p, pl.ds(i * 16, 16)]
            r, col = i // 8, (i % 8) * 16
            dsti[p, r, pl.ds(col, 16)] = d

    def fire_gathers(p):
        pltpu.async_copy(xw_hbm.at[fsrc.at[p]], rows.at[p], sem_g)
        pltpu.async_copy(qn_hbm.at[fdst.at[p]], qd.at[p], sem_g)
        pltpu.async_copy(kn_hbm.at[fsrc.at[p]], ks.at[p], sem_g)

    def drain_gathers(p):
        pltpu.make_async_copy(xw_hbm.at[pl.ds(0, CH)], rows.at[p],
                              sem_g).wait()
        pltpu.make_async_copy(qn_hbm.at[pl.ds(0, CH)], qd.at[p],
                              sem_g).wait()
        pltpu.make_async_copy(kn_hbm.at[pl.ds(0, CH)], ks.at[p],
                              sem_g).wait()

    def fire_scatters(p):
        for k in range(KSUB):
            pltpu.async_copy(rows.at[p, pl.ds(k * 128, 128), :],
                             accv_sh.at[dsti.at[p, k]], sem_s, add=True)
            pltpu.async_copy(wb.at[p, pl.ds(k * 128, 128)],
                             accs_sh.at[dsti.at[p, k]], sem_s, add=True)

    def drain_scatters(p):
        for k in range(KSUB):
            pltpu.make_async_copy(rows.at[p, pl.ds(k * 128, 128), :],
                                  accv_sh.at[pl.ds(0, 128), :], sem_s).wait()
            pltpu.make_async_copy(wb.at[p, pl.ds(k * 128, 128)],
                                  accs_sh.at[pl.ds(0, 128)], sem_s).wait()

    def w_scale(p):
        iota16 = lax.broadcasted_iota(jnp.int32, (16,), 0)

        @plsc.parallel_loop(0, CH // 16, unroll=2)
        def _(i):
            a = qd[p, pl.ds(i * 16, 16)] + ks[p, pl.ds(i * 16, 16)]
            a = jnp.where(a >= 0.0, a, 0.2 * a) - cbv
            wv = jnp.exp(a)
            wb[p, pl.ds(i * 16, 16)] = wv
            for j in range(16):
                wj = wv[j]
                r = i * 16 + j
                rows[p, r, pl.ds(0, 16)] = rows[p, r, pl.ds(0, 16)] * wj
                rows[p, r, pl.ds(16, 16)] = rows[p, r, pl.ds(16, 16)] * wj

    fire_idxcopy(0, 0)
    drain_idxcopy(0)
    idx_gather(0)
    idx_scatter(0)
    fire_gathers(0)
    fire_idxcopy(1, 1)

    @pl.loop(0, nt)
    def _pair(t):
        a = 2 * t

        @pl.when(t > 0)
        def _():
            drain_scatters(1)

        drain_idxcopy(1)
        idx_gather(1)
        idx_scatter(1)

        @pl.when(t < nt - 1)
        def _():
            fire_idxcopy(a + 2, 0)

        drain_gathers(0)
        fire_gathers(1)
        w_scale(0)
        fire_scatters(0)

        @pl.when(t < nt - 1)
        def _():
            drain_idxcopy(0)
            idx_gather(0)

        drain_gathers(1)
        drain_scatters(0)

        @pl.when(t < nt - 1)
        def _():
            idx_scatter(0)
            fire_gathers(0)
            fire_idxcopy(a + 3, 1)

        w_scale(1)
        fire_scatters(1)

    drain_scatters(1)
    plsc.subcore_barrier()
    pltpu.sync_copy(accv_sh.at[pl.ds(sid * RPT, RPT), :],
                    outv_hbm.at[cid, pl.ds(sid * RPT, RPT), :])
    pltpu.sync_copy(accs_sh.at[pl.ds(sid * RPT, RPT)],
                    outs_hbm.at[cid, pl.ds(sid * RPT, RPT)])


_EDGE = None


def _get_edge_kernel():
    global _EDGE
    if _EDGE is None:
        _EDGE = functools.partial(
            pl.kernel,
            out_type=(jax.ShapeDtypeStruct((2, NACC, HID), jnp.float32),
                      jax.ShapeDtypeStruct((2, NACC), jnp.float32)),
            mesh=plsc.VectorSubcoreMesh(core_axis_name="c",
                                        subcore_axis_name="s",
                                        num_cores=2, num_subcores=16),
            compiler_params=pltpu.CompilerParams(use_tc_tiling_on_sc=False,
                                                 needs_layout_passes=False),
            scratch_types=[
                pltpu.VMEM_SHARED((NACC, HID), jnp.float32),
                pltpu.VMEM_SHARED((NACC,), jnp.float32),
                pltpu.VMEM((2, CH), jnp.int32),
                pltpu.VMEM((2, CH), jnp.int32),
                pltpu.VMEM((2, CH), jnp.int32),
                pltpu.VMEM((2, CH), jnp.int32),
                pltpu.VMEM((2, CH), jnp.int32),
                pltpu.VMEM((2, KSUB, 128), jnp.int32),
                pltpu.VMEM((2, CH, HID), jnp.float32),
                pltpu.VMEM((2, CH), jnp.float32),
                pltpu.VMEM((2, CH), jnp.float32),
                pltpu.VMEM((2, CH), jnp.float32),
                pltpu.VMEM((16,), jnp.float32),
                pltpu.SemaphoreType.DMA,
                pltpu.SemaphoreType.DMA,
                pltpu.SemaphoreType.DMA,
            ],
        )(_edge_body)
    return _EDGE


def _finish_body(av_ref, as_ref, b_ref, prev_ref, bnw_ref, bnb_ref, x_ref):
    s = av_ref[0] + av_ref[1]
    denom = as_ref[0] + as_ref[1]
    h = s * (1.0 / (denom + 1e-16)) + b_ref[:] + prev_ref[:]
    cnt = 1.0 / (N * HID)
    m = jnp.sum(h) * cnt
    xc = h - m
    v = jnp.sum(xc * xc) * cnt
    xn = xc * (1.0 / (jnp.sqrt(v) + EPS)) * bnw_ref[:] + bnb_ref[:]
    x_ref[:] = xn * (1.0 / (1.0 + jnp.exp(-xn)))


_finish = pl.pallas_call(
    _finish_body,
    out_shape=jax.ShapeDtypeStruct((N, HID), jnp.float32),
)


def _epilogue_body(x_ref, w_ref, b_ref, o_ref):
    x3 = jnp.sum(x_ref[:], axis=0, keepdims=True) * (1.0 / N)
    y = lax.dot_general(x3, w_ref[:], (((1,), (0,)), ((), ())),
                        preferred_element_type=jnp.float32) + b_ref[:]
    o_ref[:] = y * (1.0 / (1.0 + jnp.exp(-y)))


_epilogue = pl.pallas_call(
    _epilogue_body,
    out_shape=jax.ShapeDtypeStruct((1, 256), jnp.float32),
)


def _pack_weights(params):
    packed = {}
    for i in range(NLAYERS):
        W = params['conv%d_w' % i]
        D = W.shape[1]
        qv = jnp.einsum('rdo,o->dr', W, params['conv%d_q' % i][:, 0])
        kv = jnp.einsum('rdo,o->dr', W, params['conv%d_k' % i][:, 0])
        Wt = jnp.transpose(W, (1, 0, 2))
        packed['cw%d' % i] = Wt.reshape(D, WB)
        cqk = jnp.zeros((D, 32), jnp.float32)
        cqk = cqk.at[:, :NREL].set(qv).at[:, 16:16 + NREL].set(kv)
        packed['cqk%d' % i] = cqk
        packed['b%d' % i] = params['conv%d_b' % i].reshape(1, HID)
        packed['bnw%d' % i] = params['bn%d_w' % i].reshape(1, HID)
        packed['bnb%d' % i] = params['bn%d_b' % i].reshape(1, HID)
    packed['lnw'] = jnp.pad(params['ln1_w'], (3, 0)).reshape(1, IN_DIM)
    packed['lnb'] = jnp.pad(params['ln1_b'], (3, 0)).reshape(1, IN_DIM)
    packed['linw'] = params['lin1_w'].T
    packed['linb'] = params['lin1_b'].reshape(1, 256)
    return packed


_PREP = None


def _get_preps():
    global _PREP
    if _PREP is None:
        _PREP = (_make_prep0(), _make_prep(HID))
    return _PREP


def kernel(data_batch_1, data_batch_2, edge_index_1, edge_index_2,
           edge_type_1, edge_type_2, params):
    del data_batch_2, edge_index_2, edge_type_2
    pk = _pack_weights(params)
    prep0, prep = _get_preps()

    pos_s, fln, C = _prologue(data_batch_1, pk['lnw'], pk['lnb'])
    _, v = jnp.linalg.eigh(C[:3, :3])
    vpad = jnp.zeros((IN_DIM, IN_DIM), jnp.float32).at[:3, :3].set(v)

    src = jnp.pad(edge_index_1[0], (0, EP - E))
    dst = jnp.pad(edge_index_1[1], (0, EP - E), constant_values=N)
    et = jnp.pad(edge_type_1, (0, EP - E))
    zv = jnp.zeros((NACC, HID), jnp.float32)
    zs = jnp.zeros((NACC,), jnp.float32)

    x = None
    for i in range(NLAYERS):
        if i == 0:
            y1, y2, mx = prep0(pos_s, fln, vpad, pk['cw0'], pk['cqk0'])
        else:
            y1, y2, mx = prep(x, pk['cw%d' % i], pk['cqk%d' % i])
        xw = y1.reshape(N * NREL, HID)
        qn = jnp.concatenate([y2[:, :NREL].reshape(N * NREL),
                              jnp.full((16,), -1e30, jnp.float32)])
        kn = y2[:, 16:16 + NREL].reshape(N * NREL)
        z = jnp.max(mx[0, :NREL]) + jnp.max(mx[0, 16:16 + NREL])
        cb = jnp.where(z >= 0.0, z, 0.2 * z)
        cb16 = jnp.broadcast_to(cb.reshape(1), (16,))
        outv, outs = _get_edge_kernel()(xw, qn, kn, src, dst, et, cb16,
                                        zv, zs)
        prev = x if x is not None else jnp.zeros((N, HID), jnp.float32)
        x = _finish(outv[:, :N, :], outs[:, :N, None], pk['b%d' % i], prev,
                    pk['bnw%d' % i], pk['bnb%d' % i])

    out = _epilogue(x, pk['linw'], pk['linb'])
    return out, out
